# baseline (speedup 1.0000x reference)
import functools

import jax
import jax.numpy as jnp
from jax import lax
from jax.experimental import pallas as pl
from jax.experimental.pallas import tpu as pltpu
from jax.experimental.pallas import tpu_sc as plsc

_NUMS = (3, 4, 5, 6, 7, 8, 9, 14, 15, 16, 17, 18, 22, 23, 24, 25, 26, 27,
         28, 29, 30, 31, 37, 38, 39, 46, 47, 48, 49, 50, 51, 52, 53, 54,
         55, 56, 57)

def _signed32(u):
    return u - (1 << 32) if u >= (1 << 31) else u

_MASK_LO = _signed32(sum(1 << n for n in _NUMS if n < 32))
_MASK_HI = _signed32(sum(1 << (n - 32) for n in _NUMS if n >= 32))

_NC = 2
_NS = 16
_NW = _NC * _NS
_L = 16

_ROWS = 64
_COLS = 32768
_RPW = _ROWS // _NW
_CH = 8192
_CPR = _COLS // _CH
_NCHUNK = _RPW * _CPR
_NBUF = 3


def _sc_body(x_hbm, out_hbm, *scratch):
    ins = scratch[0:_NBUF]
    outs = scratch[_NBUF:2 * _NBUF]
    isems = scratch[2 * _NBUF:3 * _NBUF]
    osems = scratch[3 * _NBUF:4 * _NBUF]
    wid = lax.axis_index("s") * _NC + lax.axis_index("c")
    row0 = wid * _RPW
    lo_vec = jnp.full((_L,), _MASK_LO, jnp.int32)
    hi_vec = jnp.full((_L,), _MASK_HI, jnp.int32)

    def _slc(ref, c):
        return ref.at[row0 + c // _CPR, pl.ds((c % _CPR) * _CH, _CH)]

    def _in_copy(c):
        return pltpu.async_copy(_slc(x_hbm, c), ins[c % _NBUF], isems[c % _NBUF])

    def _out_copy(c):
        return pltpu.async_copy(outs[c % _NBUF], _slc(out_hbm, c), osems[c % _NBUF])

    h_in = {c: _in_copy(c) for c in range(_NBUF)}
    h_out = {}
    for c in range(_NCHUNK):
        h_in[c].wait()
        if c >= _NBUF:
            h_out[c - _NBUF].wait()
        src = ins[c % _NBUF]
        dst = outs[c % _NBUF]

        @plsc.parallel_loop(0, _CH, step=_L, unroll=8)
        def _compute(i):
            v = src[pl.ds(i, _L)]
            word = jnp.where(v < 32, lo_vec, hi_vec)
            bit = lax.shift_right_logical(word, v & 31) & 1
            dst[pl.ds(i, _L)] = lax.shift_left(v, bit)

        h_out[c] = _out_copy(c)
        if c + _NBUF < _NCHUNK:
            h_in[c + _NBUF] = _in_copy(c + _NBUF)
    for c in range(_NCHUNK - _NBUF, _NCHUNK):
        h_out[c].wait()


@functools.cache
def _sc_call():
    return functools.partial(
        pl.kernel,
        out_type=jax.ShapeDtypeStruct((_ROWS, _COLS), jnp.int32),
        mesh=plsc.VectorSubcoreMesh(
            core_axis_name="c", subcore_axis_name="s",
            num_cores=_NC, num_subcores=_NS),
        scratch_types=(
            [pltpu.VMEM((_CH,), jnp.int32) for _ in range(2 * _NBUF)]
            + [pltpu.SemaphoreType.DMA for _ in range(2 * _NBUF)]
        ),
    )(_sc_body)


@jax.jit
def kernel(x):
    return _sc_call()(x)
